# Initial kernel scaffold; baseline (speedup 1.0000x reference)
import jax
import jax.numpy as jnp
from jax.experimental import pallas as pl


def kernel(input_ids, top_k, W_emb, W1, b1, W2, b2, temperature):
    raise NotImplementedError("write your pallas kernel here")


# speedup vs baseline: 1.9840x; 1.9840x over previous
import functools

import jax
import jax.numpy as jnp
from jax import lax
from jax.experimental import pallas as pl
from jax.experimental.pallas import tpu as pltpu
from jax.experimental.pallas import tpu_sc as plsc

_NC = 2
_NS = 16
_L = 16
_CH = 128


def _make_pool(B, T, D):
  nw = _NC * _NS
  b_per_w = B // nw
  n_ch = T // _CH
  nd = D // _L
  mesh = plsc.VectorSubcoreMesh(core_axis_name="c", subcore_axis_name="s")

  @functools.partial(
      pl.kernel,
      out_type=jax.ShapeDtypeStruct((B, D), jnp.float32),
      mesh=mesh,
      scratch_types=[
          pltpu.VMEM((T,), jnp.int32),
          pltpu.VMEM((2, _CH, D), jnp.float32),
          pltpu.VMEM((D,), jnp.float32),
          pltpu.SemaphoreType.DMA,
          pltpu.SemaphoreType.DMA,
      ],
  )
  def pool(ids_hbm, table_hbm, out_hbm, idx_v, rows_v, acc_v, sem0, sem1):
    wid = lax.axis_index("s") * _NC + lax.axis_index("c")
    sems = (sem0, sem1)

    def start(c, buf):
      return pltpu.async_copy(
          table_hbm.at[idx_v.at[pl.ds(c * _CH, _CH)]], rows_v.at[buf],
          sems[buf])

    def accum(buf, accs):
      def body(r, accs):
        return tuple(
            accs[d] + rows_v[buf, r, pl.ds(d * _L, _L)] for d in range(nd))
      return lax.fori_loop(0, _CH, body, accs)

    def do_row(b, carry):
      row = wid * b_per_w + b
      pltpu.sync_copy(ids_hbm.at[row], idx_v)
      accs = tuple(jnp.zeros((_L,), jnp.float32) for _ in range(nd))
      cps = [start(0, 0)]
      for c in range(n_ch):
        if c + 1 < n_ch:
          cps.append(start(c + 1, (c + 1) % 2))
        cps[c].wait()
        accs = accum(c % 2, accs)
      inv = jnp.float32(1.0 / T)
      for d in range(nd):
        acc_v[pl.ds(d * _L, _L)] = accs[d] * inv
      pltpu.sync_copy(acc_v, out_hbm.at[row])
      return carry

    lax.fori_loop(0, b_per_w, do_row, 0)

  return pool


def _mlp_body(x_ref, w1_ref, b1_ref, w2_ref, b2_ref, scale_ref, flag_ref,
              out_ref):
  x = x_ref[...]
  h = jnp.dot(x, w1_ref[...], preferred_element_type=jnp.float32) + b1_ref[...]
  h = 0.5 * h * (1.0 + lax.erf(h * jnp.float32(0.7071067811865476)))
  logits = (jnp.dot(h, w2_ref[...], preferred_element_type=jnp.float32)
            + b2_ref[...])
  logits = logits * scale_ref[...]
  bq, m = logits.shape
  idx = lax.broadcasted_iota(jnp.int32, (bq, m), 1)
  work = logits
  mask = jnp.zeros_like(logits)
  for _ in range(8):
    mx = jnp.max(work, axis=1, keepdims=True)
    cand = work == mx
    cidx = jnp.where(cand, idx, m)
    pick = jnp.min(cidx, axis=1, keepdims=True)
    sel = idx == pick
    mask = jnp.where(sel, 1.0, mask)
    work = jnp.where(sel, -jnp.inf, work)
  masked = logits * mask + (1.0 - mask) * jnp.float32(-1e9)
  l = jnp.where(flag_ref[...] > 0, masked, logits)
  mx2 = jnp.max(l, axis=1, keepdims=True)
  e = jnp.exp(l - mx2)
  out_ref[...] = e / jnp.sum(e, axis=1, keepdims=True)


def kernel(input_ids, top_k, W_emb, W1, b1, W2, b2, temperature):
  B, T = input_ids.shape
  V, D = W_emb.shape
  N = W2.shape[1]
  ids = input_ids.astype(jnp.int32)
  x = _make_pool(B, T, D)(ids, W_emb)
  tk = jnp.asarray(top_k)
  flag = ((tk > 0) & (tk < N)).astype(jnp.float32)
  scale = 1.0 / jnp.maximum(jnp.abs(temperature.astype(jnp.float32)),
                            jnp.float32(0.1))
  scale_row = jnp.full((1, N), scale, dtype=jnp.float32)
  flag_row = jnp.full((1, N), flag, dtype=jnp.float32)
  weights = pl.pallas_call(
      _mlp_body,
      out_shape=jax.ShapeDtypeStruct((B, N), jnp.float32),
  )(x, W1, b1.reshape(1, -1), W2, b2.reshape(1, -1), scale_row, flag_row)
  return weights


# speedup vs baseline: 2.1926x; 1.1052x over previous
import functools

import jax
import jax.numpy as jnp
from jax import lax
from jax.experimental import pallas as pl
from jax.experimental.pallas import tpu as pltpu
from jax.experimental.pallas import tpu_sc as plsc

_NC = 2
_NS = 16
_L = 16
_CH = 128


def _make_pool(B, T, D):
  nw = _NC * _NS
  b_per_w = B // nw
  n_ch = T // _CH
  nd = D // _L
  mesh = plsc.VectorSubcoreMesh(core_axis_name="c", subcore_axis_name="s")

  nbuf = 3

  @functools.partial(
      pl.kernel,
      out_type=jax.ShapeDtypeStruct((B, D), jnp.float32),
      mesh=mesh,
      scratch_types=[
          pltpu.VMEM((b_per_w, T), jnp.int32),
          pltpu.VMEM((nbuf, _CH, D), jnp.float32),
          pltpu.VMEM((D,), jnp.float32),
          pltpu.SemaphoreType.DMA,
          pltpu.SemaphoreType.DMA,
          pltpu.SemaphoreType.DMA,
      ],
  )
  def pool(ids_hbm, table_hbm, out_hbm, idx_v, rows_v, acc_v, sem0, sem1,
           sem2):
    sems = (sem0, sem1, sem2)
    wid = lax.axis_index("s") * _NC + lax.axis_index("c")
    pltpu.sync_copy(ids_hbm.at[pl.ds(wid * b_per_w, b_per_w)], idx_v)

    def start(b, c, buf):
      pltpu.async_copy(
          table_hbm.at[idx_v.at[b, pl.ds(c * _CH, _CH)]], rows_v.at[buf],
          sems[buf])

    def wait(buf):
      pltpu.make_async_copy(
          table_hbm.at[idx_v.at[0, pl.ds(0, _CH)]], rows_v.at[buf],
          sems[buf]).wait()

    def accum(buf, accs):
      def body(r2, accs):
        r = r2 * 2
        accs = tuple(
            accs[d] + rows_v[buf, r, pl.ds(d * _L, _L)] for d in range(nd))
        return tuple(
            accs[d] + rows_v[buf, r + 1, pl.ds(d * _L, _L)]
            for d in range(nd))
      return lax.fori_loop(0, _CH // 2, body, accs)

    def do_row(b, carry):
      row = wid * b_per_w + b
      accs = tuple(jnp.zeros((_L,), jnp.float32) for _ in range(nd))
      for c in range(nbuf - 1):
        start(b, c, c)
      for c in range(n_ch):
        if c + nbuf - 1 < n_ch:
          start(b, c + nbuf - 1, (c + nbuf - 1) % nbuf)
        wait(c % nbuf)
        accs = accum(c % nbuf, accs)
      inv = jnp.float32(1.0 / T)
      for d in range(nd):
        acc_v[pl.ds(d * _L, _L)] = accs[d] * inv
      pltpu.sync_copy(acc_v, out_hbm.at[row])
      return carry

    lax.fori_loop(0, b_per_w, do_row, 0)

  return pool


def _mlp_body(x_ref, w1_ref, b1_ref, w2_ref, b2_ref, scale_ref, flag_ref,
              out_ref):
  x = x_ref[...]
  h = jnp.dot(x, w1_ref[...], preferred_element_type=jnp.float32) + b1_ref[...]
  h = 0.5 * h * (1.0 + lax.erf(h * jnp.float32(0.7071067811865476)))
  logits = (jnp.dot(h, w2_ref[...], preferred_element_type=jnp.float32)
            + b2_ref[...])
  logits = logits * scale_ref[...]
  bq, m = logits.shape
  idx = lax.broadcasted_iota(jnp.int32, (bq, m), 1)
  work = logits
  mask = jnp.zeros_like(logits)
  for _ in range(8):
    mx = jnp.max(work, axis=1, keepdims=True)
    cand = work == mx
    cidx = jnp.where(cand, idx, m)
    pick = jnp.min(cidx, axis=1, keepdims=True)
    sel = idx == pick
    mask = jnp.where(sel, 1.0, mask)
    work = jnp.where(sel, -jnp.inf, work)
  masked = logits * mask + (1.0 - mask) * jnp.float32(-1e9)
  l = jnp.where(flag_ref[...] > 0, masked, logits)
  mx2 = jnp.max(l, axis=1, keepdims=True)
  e = jnp.exp(l - mx2)
  out_ref[...] = e / jnp.sum(e, axis=1, keepdims=True)


def kernel(input_ids, top_k, W_emb, W1, b1, W2, b2, temperature):
  B, T = input_ids.shape
  V, D = W_emb.shape
  N = W2.shape[1]
  ids = input_ids.astype(jnp.int32)
  x = _make_pool(B, T, D)(ids, W_emb)
  tk = jnp.asarray(top_k)
  flag = ((tk > 0) & (tk < N)).astype(jnp.float32)
  scale = 1.0 / jnp.maximum(jnp.abs(temperature.astype(jnp.float32)),
                            jnp.float32(0.1))
  scale_row = jnp.full((1, N), scale, dtype=jnp.float32)
  flag_row = jnp.full((1, N), flag, dtype=jnp.float32)
  weights = pl.pallas_call(
      _mlp_body,
      out_shape=jax.ShapeDtypeStruct((B, N), jnp.float32),
  )(x, W1, b1.reshape(1, -1), W2, b2.reshape(1, -1), scale_row, flag_row)
  return weights


# speedup vs baseline: 2.5173x; 1.1481x over previous
import functools

import jax
import jax.numpy as jnp
from jax import lax
from jax.experimental import pallas as pl
from jax.experimental.pallas import tpu as pltpu
from jax.experimental.pallas import tpu_sc as plsc

_NC = 2
_NS = 16
_L = 16
_CH = 128


def _make_pool(B, T, D):
  ch = 64
  nw = _NC * _NS
  b_per_w = B // nw
  cpr = T // ch
  total = cpr * b_per_w
  sh = cpr.bit_length() - 1
  nd = D // _L
  nbuf = 4
  mesh = plsc.VectorSubcoreMesh(core_axis_name="c", subcore_axis_name="s")

  @functools.partial(
      pl.kernel,
      out_type=jax.ShapeDtypeStruct((B, D), jnp.float32),
      mesh=mesh,
      scratch_types=[
          pltpu.VMEM((b_per_w, T), jnp.int32),
          pltpu.VMEM((nbuf, ch, D), jnp.float32),
          pltpu.VMEM((D,), jnp.float32),
          pltpu.SemaphoreType.DMA,
          pltpu.SemaphoreType.DMA,
          pltpu.SemaphoreType.DMA,
          pltpu.SemaphoreType.DMA,
      ],
  )
  def pool(ids_hbm, table_hbm, out_hbm, idx_v, rows_v, acc_v, sem0, sem1,
           sem2, sem3):
    sems = (sem0, sem1, sem2, sem3)
    wid = lax.axis_index("s") * _NC + lax.axis_index("c")
    pltpu.sync_copy(ids_hbm.at[pl.ds(wid * b_per_w, b_per_w)], idx_v)

    def start(g, buf):
      row = lax.shift_right_logical(g, sh)
      col = lax.bitwise_and(g, cpr - 1)
      pltpu.async_copy(
          table_hbm.at[idx_v.at[row, pl.ds(col * ch, ch)]], rows_v.at[buf],
          sems[buf])

    def wait(buf):
      pltpu.make_async_copy(
          table_hbm.at[idx_v.at[0, pl.ds(0, ch)]], rows_v.at[buf],
          sems[buf]).wait()

    def accum(buf, accs):
      def body(r2, accs):
        r = r2 * 2
        accs = tuple(
            accs[d] + rows_v[buf, r, pl.ds(d * _L, _L)] for d in range(nd))
        return tuple(
            accs[d] + rows_v[buf, r + 1, pl.ds(d * _L, _L)]
            for d in range(nd))
      return lax.fori_loop(0, ch // 2, body, accs)

    def process(g, buf, accs, do_start):
      wait(buf)
      accs = accum(buf, accs)
      if do_start:
        start(g + nbuf, buf)
      boundary = lax.bitwise_and(g, cpr - 1) == (cpr - 1)
      row = lax.shift_right_logical(g, sh)

      @pl.when(boundary)
      def _():
        inv = jnp.float32(1.0 / T)
        for d in range(nd):
          acc_v[pl.ds(d * _L, _L)] = accs[d] * inv
        pltpu.sync_copy(acc_v, out_hbm.at[wid * b_per_w + row])

      keep = jnp.where(boundary, jnp.float32(0.0), jnp.float32(1.0))
      return tuple(a * keep for a in accs)

    for b in range(nbuf):
      start(jnp.int32(b), b)
    accs0 = tuple(jnp.zeros((_L,), jnp.float32) for _ in range(nd))

    def outer(g4, accs):
      g = g4 * nbuf
      for j in range(nbuf):
        accs = process(g + j, j, accs, True)
      return accs

    accs = lax.fori_loop(0, total // nbuf - 1, outer, accs0)
    for j in range(nbuf):
      accs = process(jnp.int32(total - nbuf + j), j, accs, False)

  return pool


def _mlp_body(x_ref, w1_ref, b1_ref, w2_ref, b2_ref, scale_ref, flag_ref,
              out_ref):
  x = x_ref[...]
  h = jnp.dot(x, w1_ref[...], preferred_element_type=jnp.float32) + b1_ref[...]
  h = 0.5 * h * (1.0 + lax.erf(h * jnp.float32(0.7071067811865476)))
  logits = (jnp.dot(h, w2_ref[...], preferred_element_type=jnp.float32)
            + b2_ref[...])
  logits = logits * scale_ref[...]
  bq, m = logits.shape
  idx = lax.broadcasted_iota(jnp.int32, (bq, m), 1)
  work = logits
  mask = jnp.zeros_like(logits)
  for _ in range(8):
    mx = jnp.max(work, axis=1, keepdims=True)
    cand = work == mx
    cidx = jnp.where(cand, idx, m)
    pick = jnp.min(cidx, axis=1, keepdims=True)
    sel = idx == pick
    mask = jnp.where(sel, 1.0, mask)
    work = jnp.where(sel, -jnp.inf, work)
  masked = logits * mask + (1.0 - mask) * jnp.float32(-1e9)
  l = jnp.where(flag_ref[...] > 0, masked, logits)
  mx2 = jnp.max(l, axis=1, keepdims=True)
  e = jnp.exp(l - mx2)
  out_ref[...] = e / jnp.sum(e, axis=1, keepdims=True)


def kernel(input_ids, top_k, W_emb, W1, b1, W2, b2, temperature):
  B, T = input_ids.shape
  V, D = W_emb.shape
  N = W2.shape[1]
  ids = input_ids.astype(jnp.int32)
  x = _make_pool(B, T, D)(ids, W_emb)
  tk = jnp.asarray(top_k)
  flag = ((tk > 0) & (tk < N)).astype(jnp.float32)
  scale = 1.0 / jnp.maximum(jnp.abs(temperature.astype(jnp.float32)),
                            jnp.float32(0.1))
  scale_row = jnp.full((1, N), scale, dtype=jnp.float32)
  flag_row = jnp.full((1, N), flag, dtype=jnp.float32)
  weights = pl.pallas_call(
      _mlp_body,
      out_shape=jax.ShapeDtypeStruct((B, N), jnp.float32),
  )(x, W1, b1.reshape(1, -1), W2, b2.reshape(1, -1), scale_row, flag_row)
  return weights
